# Initial kernel scaffold; baseline (speedup 1.0000x reference)
import jax
import jax.numpy as jnp
from jax.experimental import pallas as pl


def kernel(y_pred):
    raise NotImplementedError("write your pallas kernel here")


# speedup vs baseline: 21.5259x; 21.5259x over previous
import jax
import jax.numpy as jnp
from jax.experimental import pallas as pl
from jax.experimental.pallas import tpu as pltpu

_CONF_T = 0.01
_IOU_T = 0.45
_TOPK = 200
_KSEL = 200
_B = 4
_N = 8732
_NCLS = 20
_L = _B * _NCLS
_W = 8960
_IMG_H = 300.0
_IMG_W = 300.0


def _nms_body(scores_ref, x1_ref, y1_ref, x2_ref, y2_ref,
              cls_out, conf_out, ox1_out, oy1_out, ox2_out, oy2_out,
              work_ref, area_ref):
    scores = scores_ref[...]
    work_ref[...] = jnp.where(scores > _CONF_T, scores, -jnp.inf)
    x1 = x1_ref[...]
    y1 = y1_ref[...]
    x2 = x2_ref[...]
    y2 = y2_ref[...]
    area_ref[...] = (jnp.maximum(x2 - x1, 0.0) * jnp.maximum(y2 - y1, 0.0))

    iota = jax.lax.broadcasted_iota(jnp.int32, (_L, _W), 1)
    lane = jax.lax.broadcasted_iota(jnp.int32, (_L, 1), 0)
    clsvec = (lane % _NCLS + 1).astype(jnp.float32)
    col = jax.lax.broadcasted_iota(jnp.int32, (_L, _KSEL), 1)

    def body(k, _):
        work = work_ref[...]
        m = jnp.max(work, axis=1, keepdims=True)
        msk = work == m
        idx = jnp.min(jnp.where(msk, iota, _W), axis=1, keepdims=True)
        onehot = iota == idx

        x1 = x1_ref[...]
        y1 = y1_ref[...]
        x2 = x2_ref[...]
        y2 = y2_ref[...]
        zero = jnp.zeros_like(x1)
        sx1 = jnp.sum(jnp.where(onehot, x1, zero), axis=1, keepdims=True)
        sy1 = jnp.sum(jnp.where(onehot, y1, zero), axis=1, keepdims=True)
        sx2 = jnp.sum(jnp.where(onehot, x2, zero), axis=1, keepdims=True)
        sy2 = jnp.sum(jnp.where(onehot, y2, zero), axis=1, keepdims=True)

        xi1 = jnp.maximum(sx1, x1)
        yi1 = jnp.maximum(sy1, y1)
        xi2 = jnp.minimum(sx2, x2)
        yi2 = jnp.minimum(sy2, y2)
        inter = jnp.maximum(xi2 - xi1, 0.0) * jnp.maximum(yi2 - yi1, 0.0)
        a1 = jnp.maximum(sx2 - sx1, 0.0) * jnp.maximum(sy2 - sy1, 0.0)
        iou = inter / (a1 + area_ref[...] - inter + 1e-8)

        supp = (iou >= _IOU_T) | onehot
        work_ref[...] = jnp.where(supp, -jnp.inf, work)

        ok = m > 0.0
        z1 = jnp.zeros_like(m)
        here = col == k
        for ref, val in ((cls_out, clsvec), (conf_out, m),
                         (ox1_out, sx1), (oy1_out, sy1),
                         (ox2_out, sx2), (oy2_out, sy2)):
            v = jnp.where(ok, val, z1)
            ref[...] = jnp.where(here, v, ref[...])
        return 0

    jax.lax.fori_loop(0, _KSEL, body, 0)


def _decode_boxes(y_pred):
    cx = y_pred[..., -12] * y_pred[..., -4] * y_pred[..., -6] + y_pred[..., -8]
    cy = y_pred[..., -11] * y_pred[..., -3] * y_pred[..., -5] + y_pred[..., -7]
    w = jnp.exp(y_pred[..., -10] * y_pred[..., -2]) * y_pred[..., -6]
    h = jnp.exp(y_pred[..., -9] * y_pred[..., -1]) * y_pred[..., -5]
    xmin = (cx - 0.5 * w) * _IMG_W
    ymin = (cy - 0.5 * h) * _IMG_H
    xmax = (cx + 0.5 * w) * _IMG_W
    ymax = (cy + 0.5 * h) * _IMG_H
    return xmin, ymin, xmax, ymax


def kernel(y_pred):
    xmin, ymin, xmax, ymax = _decode_boxes(y_pred)
    confs = y_pred[..., 1:_NCLS + 1]

    pad = _W - _N
    scores = jnp.transpose(confs, (0, 2, 1))
    scores = jnp.pad(scores, ((0, 0), (0, 0), (0, pad)))
    scores = scores.reshape(_L, _W)

    def lanes(a):
        a = jnp.pad(a, ((0, 0), (0, pad)))
        return jnp.broadcast_to(a[:, None, :], (_B, _NCLS, _W)).reshape(_L, _W)

    out_sd = [jax.ShapeDtypeStruct((_L, _KSEL), jnp.float32)] * 6
    outs = pl.pallas_call(
        _nms_body,
        out_shape=out_sd,
        scratch_shapes=[
            pltpu.VMEM((_L, _W), jnp.float32),
            pltpu.VMEM((_L, _W), jnp.float32),
        ],
    )(scores, lanes(xmin), lanes(ymin), lanes(xmax), lanes(ymax))

    cls_r, conf_r, x1_r, y1_r, x2_r, y2_r = outs

    flat_conf = conf_r.reshape(_B, _NCLS * _KSEL)
    _, top_idx = jax.lax.top_k(flat_conf, _TOPK)
    fields = [cls_r, conf_r, x1_r, y1_r, x2_r, y2_r]
    gathered = [jnp.take_along_axis(f.reshape(_B, _NCLS * _KSEL), top_idx, axis=1)
                for f in fields]
    return jnp.stack(gathered, axis=-1)


# speedup vs baseline: 37.0401x; 1.7207x over previous
import functools

import jax
import jax.numpy as jnp
from jax import lax
from jax.experimental import pallas as pl
from jax.experimental.pallas import tpu as pltpu
from jax.experimental.pallas import tpu_sc as plsc

_CONF_T = 0.01
_IOU_T = 0.45
_TOPK = 200
_KSEL = 200
_B = 4
_N = 8732
_NCLS = 20
_L = _B * _NCLS
_W = 8960
_C = 2048
_IMG_H = 300.0
_IMG_W = 300.0

_KMAX = 0x3F7FFFFF
_NV = _W // 16
_NVC = _C // 16
_NW = 32


def _sc_sort_body(keys_hbm, x1_hbm, y1_hbm, x2_hbm, y2_hbm,
                  okey, ox1, oy1, ox2, oy2, ocnt,
                  kA, iA, kB, iB, vx1, vy1, vx2, vy2,
                  hist, base, s1, s2, s3, s4, scnt):
    cid = lax.axis_index("c")
    sid = lax.axis_index("s")
    wid = sid * 2 + cid

    def process(lane):
        batch = ((lane >= _NCLS).astype(jnp.int32)
                 + (lane >= 2 * _NCLS) + (lane >= 3 * _NCLS))
        pltpu.sync_copy(keys_hbm.at[lane], kA)

        kthr = _KMAX - 0x3C23D70A
        lane16 = lax.iota(jnp.int32, 16)

        one16 = jnp.ones((16,), jnp.int32)
        zero16 = jnp.zeros((16,), jnp.int32)

        def ibody(i, acc):
            s = pl.ds(i * 16, 16)
            iA[s] = lane16 + i * 16
            return acc + jnp.sum(jnp.where(kA[s] < kthr, one16, zero16))

        acc = lax.fori_loop(0, _NV, ibody, jnp.int32(0))
        scnt[...] = jnp.broadcast_to(acc, (16,))
        pltpu.sync_copy(scnt, ocnt.at[lane])

        bufs = ((kA, iA, kB, iB), (kB, iB, kA, iA))
        for p in range(6):
            src_k, src_i, dst_k, dst_i = bufs[p % 2]
            shift = 5 * p
            z16 = jnp.zeros((16,), jnp.int32)
            hist[pl.ds(0, 16)] = z16
            hist[pl.ds(16, 16)] = z16

            def hbody(i, _, src_k=src_k, shift=shift):
                dig = (src_k[pl.ds(i * 16, 16)] >> shift) & 31
                cnt, last = plsc.scan_count(dig)
                plsc.addupdate_scatter(hist, [dig], cnt, mask=last)
                return 0

            lax.fori_loop(0, _NV, hbody, 0)

            h0 = hist[pl.ds(0, 16)]
            h1 = hist[pl.ds(16, 16)]
            c0 = plsc.cumsum(h0)
            t0 = jnp.max(c0)
            base[pl.ds(0, 16)] = c0 - h0
            base[pl.ds(16, 16)] = plsc.cumsum(h1) - h1 + t0

            def pbody(i, _, src_k=src_k, src_i=src_i,
                      dst_k=dst_k, dst_i=dst_i, shift=shift):
                s = pl.ds(i * 16, 16)
                kv = src_k[s]
                iv = src_i[s]
                dig = (kv >> shift) & 31
                cnt, last = plsc.scan_count(dig)
                pos = plsc.load_gather(base, [dig]) + cnt - 1
                plsc.store_scatter(dst_k, [pos], kv)
                plsc.store_scatter(dst_i, [pos], iv)
                plsc.addupdate_scatter(base, [dig], cnt, mask=last)
                return 0

            lax.fori_loop(0, _NV, pbody, 0)

        pltpu.sync_copy(x1_hbm.at[batch], vx1)
        pltpu.sync_copy(y1_hbm.at[batch], vy1)
        pltpu.sync_copy(x2_hbm.at[batch], vx2)
        pltpu.sync_copy(y2_hbm.at[batch], vy2)

        def gbody(j, _):
            s = pl.ds(j * 16, 16)
            iv = iA[s]
            s1[s] = plsc.load_gather(vx1, [iv])
            s2[s] = plsc.load_gather(vy1, [iv])
            s3[s] = plsc.load_gather(vx2, [iv])
            s4[s] = plsc.load_gather(vy2, [iv])
            return 0

        lax.fori_loop(0, _NVC, gbody, 0)

        pltpu.sync_copy(kA.at[pl.ds(0, _C)], okey.at[lane])
        pltpu.sync_copy(s1, ox1.at[lane])
        pltpu.sync_copy(s2, oy1.at[lane])
        pltpu.sync_copy(s3, ox2.at[lane])
        pltpu.sync_copy(s4, oy2.at[lane])

    for r in range(3):
        lane = wid + _NW * r
        if r < 2:
            process(lane)
        else:
            @pl.when(lane < _L)
            def _():
                process(lane)


@functools.partial(jax.jit, static_argnums=())
def _sc_sort(keys, bx1, by1, bx2, by2):
    mesh = plsc.VectorSubcoreMesh(core_axis_name="c", subcore_axis_name="s",
                                  num_cores=2, num_subcores=16)
    f32 = jnp.float32
    return pl.kernel(
        _sc_sort_body,
        out_type=[
            jax.ShapeDtypeStruct((_L, _C), jnp.int32),
            jax.ShapeDtypeStruct((_L, _C), f32),
            jax.ShapeDtypeStruct((_L, _C), f32),
            jax.ShapeDtypeStruct((_L, _C), f32),
            jax.ShapeDtypeStruct((_L, _C), f32),
            jax.ShapeDtypeStruct((_L, 16), jnp.int32),
        ],
        mesh=mesh,
        scratch_types=[
            pltpu.VMEM((_W,), jnp.int32),
            pltpu.VMEM((_W,), jnp.int32),
            pltpu.VMEM((_W,), jnp.int32),
            pltpu.VMEM((_W,), jnp.int32),
            pltpu.VMEM((_W,), f32),
            pltpu.VMEM((_W,), f32),
            pltpu.VMEM((_W,), f32),
            pltpu.VMEM((_W,), f32),
            pltpu.VMEM((32,), jnp.int32),
            pltpu.VMEM((32,), jnp.int32),
            pltpu.VMEM((_C,), f32),
            pltpu.VMEM((_C,), f32),
            pltpu.VMEM((_C,), f32),
            pltpu.VMEM((_C,), f32),
            pltpu.VMEM((16,), jnp.int32),
        ],
        compiler_params=pltpu.CompilerParams(needs_layout_passes=False),
    )(keys, bx1, by1, bx2, by2)


def _make_nms_body(width, with_flag):
    def body(scores_ref, x1_ref, y1_ref, x2_ref, y2_ref, *refs):
        if with_flag:
            (cls_out, conf_out, ox1_out, oy1_out, ox2_out, oy2_out,
             flag_out, work_ref, area_ref) = refs
        else:
            (cls_out, conf_out, ox1_out, oy1_out, ox2_out, oy2_out,
             work_ref, area_ref) = refs

        scores = scores_ref[...]
        work_ref[...] = jnp.where(scores > _CONF_T, scores, -jnp.inf)
        x1 = x1_ref[...]
        y1 = y1_ref[...]
        x2 = x2_ref[...]
        y2 = y2_ref[...]
        area_ref[...] = (jnp.maximum(x2 - x1, 0.0)
                         * jnp.maximum(y2 - y1, 0.0))

        iota = lax.broadcasted_iota(jnp.int32, (_L, width), 1)
        lane = lax.broadcasted_iota(jnp.int32, (_L, 1), 0)
        clsvec = (lane % _NCLS + 1).astype(jnp.float32)
        col = lax.broadcasted_iota(jnp.int32, (_L, _KSEL), 1)

        def step(k, _):
            work = work_ref[...]
            m = jnp.max(work, axis=1, keepdims=True)
            msk = work == m
            idx = jnp.min(jnp.where(msk, iota, width), axis=1, keepdims=True)
            onehot = iota == idx

            x1 = x1_ref[...]
            y1 = y1_ref[...]
            x2 = x2_ref[...]
            y2 = y2_ref[...]
            zero = jnp.zeros_like(x1)
            sx1 = jnp.sum(jnp.where(onehot, x1, zero), axis=1, keepdims=True)
            sy1 = jnp.sum(jnp.where(onehot, y1, zero), axis=1, keepdims=True)
            sx2 = jnp.sum(jnp.where(onehot, x2, zero), axis=1, keepdims=True)
            sy2 = jnp.sum(jnp.where(onehot, y2, zero), axis=1, keepdims=True)

            xi1 = jnp.maximum(sx1, x1)
            yi1 = jnp.maximum(sy1, y1)
            xi2 = jnp.minimum(sx2, x2)
            yi2 = jnp.minimum(sy2, y2)
            inter = (jnp.maximum(xi2 - xi1, 0.0)
                     * jnp.maximum(yi2 - yi1, 0.0))
            a1 = (jnp.maximum(sx2 - sx1, 0.0)
                  * jnp.maximum(sy2 - sy1, 0.0))
            iou = inter / (a1 + area_ref[...] - inter + 1e-8)

            supp = (iou >= _IOU_T) | onehot
            work_ref[...] = jnp.where(supp, -jnp.inf, work)

            ok = m > 0.0
            z1 = jnp.zeros_like(m)
            here = col == k
            for ref, val in ((cls_out, clsvec), (conf_out, m),
                             (ox1_out, sx1), (oy1_out, sy1),
                             (ox2_out, sx2), (oy2_out, sy2)):
                v = jnp.where(ok, val, z1)
                ref[...] = jnp.where(here, v, ref[...])
            if with_flag:
                flag_out[...] = jnp.broadcast_to(
                    jnp.where(ok, z1, z1 + 1.0), (_L, 128))
            return 0

        lax.fori_loop(0, _KSEL, step, 0)

    return body


def _tc_nms(scores, x1, y1, x2, y2, width, with_flag):
    f32 = jnp.float32
    out_sd = [jax.ShapeDtypeStruct((_L, _KSEL), f32)] * 6
    if with_flag:
        out_sd = out_sd + [jax.ShapeDtypeStruct((_L, 128), f32)]
    return pl.pallas_call(
        _make_nms_body(width, with_flag),
        out_shape=out_sd,
        scratch_shapes=[
            pltpu.VMEM((_L, width), f32),
            pltpu.VMEM((_L, width), f32),
        ],
    )(scores, x1, y1, x2, y2)


def _decode_boxes(y_pred):
    cx = y_pred[..., -12] * y_pred[..., -4] * y_pred[..., -6] + y_pred[..., -8]
    cy = y_pred[..., -11] * y_pred[..., -3] * y_pred[..., -5] + y_pred[..., -7]
    w = jnp.exp(y_pred[..., -10] * y_pred[..., -2]) * y_pred[..., -6]
    h = jnp.exp(y_pred[..., -9] * y_pred[..., -1]) * y_pred[..., -5]
    xmin = (cx - 0.5 * w) * _IMG_W
    ymin = (cy - 0.5 * h) * _IMG_H
    xmax = (cx + 0.5 * w) * _IMG_W
    ymax = (cy + 0.5 * h) * _IMG_H
    return xmin, ymin, xmax, ymax


def kernel(y_pred):
    xmin, ymin, xmax, ymax = _decode_boxes(y_pred)
    confs = y_pred[..., 1:_NCLS + 1]

    pad = _W - _N
    scores = jnp.transpose(confs, (0, 2, 1))
    scores = jnp.pad(scores, ((0, 0), (0, 0), (0, pad))).reshape(_L, _W)
    keys = _KMAX - lax.bitcast_convert_type(scores, jnp.int32)

    boxes_b = [jnp.pad(a, ((0, 0), (0, pad)))
               for a in (xmin, ymin, xmax, ymax)]

    skey, sx1, sy1, sx2, sy2, cnt = _sc_sort(keys, *boxes_b)
    sscores = lax.bitcast_convert_type(_KMAX - skey, jnp.float32)

    fast = _tc_nms(sscores, sx1, sy1, sx2, sy2, _C, True)
    fast_rows, flag = fast[:6], fast[6]

    need_fb = jnp.any((flag[:, 0] > 0.0) & (cnt[:, 0] > _C))

    def fallback(_):
        def lanes(a):
            return jnp.broadcast_to(
                a[:, None, :], (_B, _NCLS, _W)).reshape(_L, _W)
        full = _tc_nms(scores, *[lanes(a) for a in boxes_b], _W, False)
        return tuple(full)

    rows = lax.cond(need_fb, fallback, lambda _: tuple(fast_rows), None)

    flat_conf = rows[1].reshape(_B, _NCLS * _KSEL)
    _, top_idx = lax.top_k(flat_conf, _TOPK)
    gathered = [jnp.take_along_axis(f.reshape(_B, _NCLS * _KSEL),
                                    top_idx, axis=1)
                for f in rows]
    return jnp.stack(gathered, axis=-1)


# speedup vs baseline: 58.6256x; 1.5828x over previous
import functools

import jax
import jax.numpy as jnp
from jax import lax
from jax.experimental import pallas as pl
from jax.experimental.pallas import tpu as pltpu
from jax.experimental.pallas import tpu_sc as plsc

_CONF_T = 0.01
_IOU_T = 0.45
_TOPK = 200
_KSEL = 200
_B = 4
_N = 8732
_NCLS = 20
_L = _B * _NCLS
_W = 8960
_C = 1024
_IMG_H = 300.0
_IMG_W = 300.0

_KMAX = 0x3F7FFFFF
_KTHR = 0x3C23D70A
_KINACT = _KMAX - _KTHR
_NV = _W // 16
_NVC = _C // 16
_NW = 32


def _sc_sort_body(keys_hbm, x1_hbm, y1_hbm, x2_hbm, y2_hbm,
                  okey, ox1, oy1, ox2, oy2, ocnt,
                  kA, iA, kB, iB, vx1, vy1, vx2, vy2,
                  hist, base, s1, s2, s3, s4, scnt):
    cid = lax.axis_index("c")
    sid = lax.axis_index("s")
    wid = sid * 2 + cid

    def process(lane):
        batch = ((lane >= _NCLS).astype(jnp.int32)
                 + (lane >= 2 * _NCLS) + (lane >= 3 * _NCLS))
        pltpu.sync_copy(keys_hbm.at[lane], kA)

        kthr = _KINACT
        lane16 = lax.iota(jnp.int32, 16)

        one16 = jnp.ones((16,), jnp.int32)
        zero16 = jnp.zeros((16,), jnp.int32)

        def ibody(i, acc):
            s = pl.ds(i * 16, 16)
            iA[s] = lane16 + i * 16
            return acc + jnp.sum(jnp.where(kA[s] < kthr, one16, zero16))

        acc = lax.fori_loop(0, _NV, ibody, jnp.int32(0))
        scnt[...] = jnp.broadcast_to(acc, (16,))
        pltpu.sync_copy(scnt, ocnt.at[lane])

        z16 = jnp.zeros((16,), jnp.int32)
        bufs = ((kA, iA, kB, iB), (kB, iB, kA, iA))
        for p in range(3):
            src_k, src_i, dst_k, dst_i = bufs[p % 2]
            shift = 9 * p

            def zbody(j, _):
                hist[pl.ds(j * 16, 16)] = z16
                return 0

            lax.fori_loop(0, 32, zbody, 0)

            def hbody(i, _, src_k=src_k, shift=shift):
                dig = (src_k[pl.ds(i * 16, 16)] >> shift) & 511
                cnt, last = plsc.scan_count(dig)
                plsc.addupdate_scatter(hist, [dig], cnt, mask=last)
                return 0

            lax.fori_loop(0, _NV, hbody, 0)

            def sbody(j, carry):
                h = hist[pl.ds(j * 16, 16)]
                c = plsc.cumsum(h)
                base[pl.ds(j * 16, 16)] = c - h + carry
                return carry + jnp.max(c)

            lax.fori_loop(0, 32, sbody, jnp.int32(0))

            def pbody(i, _, src_k=src_k, src_i=src_i,
                      dst_k=dst_k, dst_i=dst_i, shift=shift):
                s = pl.ds(i * 16, 16)
                kv = src_k[s]
                iv = src_i[s]
                dig = (kv >> shift) & 511
                cnt, last = plsc.scan_count(dig)
                pos = plsc.load_gather(base, [dig]) + cnt - 1
                plsc.store_scatter(dst_k, [pos], kv)
                plsc.store_scatter(dst_i, [pos], iv)
                plsc.addupdate_scatter(base, [dig], cnt, mask=last)
                return 0

            lax.fori_loop(0, _NV, pbody, 0)

        pltpu.sync_copy(x1_hbm.at[batch], vx1)
        pltpu.sync_copy(y1_hbm.at[batch], vy1)
        pltpu.sync_copy(x2_hbm.at[batch], vx2)
        pltpu.sync_copy(y2_hbm.at[batch], vy2)

        def gbody(j, _):
            s = pl.ds(j * 16, 16)
            iv = iB[s]
            s1[s] = plsc.load_gather(vx1, [iv])
            s2[s] = plsc.load_gather(vy1, [iv])
            s3[s] = plsc.load_gather(vx2, [iv])
            s4[s] = plsc.load_gather(vy2, [iv])
            return 0

        lax.fori_loop(0, _NVC, gbody, 0)

        pltpu.sync_copy(kB.at[pl.ds(0, _C)], okey.at[lane])
        pltpu.sync_copy(s1, ox1.at[lane])
        pltpu.sync_copy(s2, oy1.at[lane])
        pltpu.sync_copy(s3, ox2.at[lane])
        pltpu.sync_copy(s4, oy2.at[lane])

    for r in range(3):
        lane = wid + _NW * r
        if r < 2:
            process(lane)
        else:
            @pl.when(lane < _L)
            def _():
                process(lane)


@functools.partial(jax.jit, static_argnums=())
def _sc_sort(keys, bx1, by1, bx2, by2):
    mesh = plsc.VectorSubcoreMesh(core_axis_name="c", subcore_axis_name="s",
                                  num_cores=2, num_subcores=16)
    f32 = jnp.float32
    return pl.kernel(
        _sc_sort_body,
        out_type=[
            jax.ShapeDtypeStruct((_L, _C), jnp.int32),
            jax.ShapeDtypeStruct((_L, _C), f32),
            jax.ShapeDtypeStruct((_L, _C), f32),
            jax.ShapeDtypeStruct((_L, _C), f32),
            jax.ShapeDtypeStruct((_L, _C), f32),
            jax.ShapeDtypeStruct((_L, 16), jnp.int32),
        ],
        mesh=mesh,
        scratch_types=[
            pltpu.VMEM((_W,), jnp.int32),
            pltpu.VMEM((_W,), jnp.int32),
            pltpu.VMEM((_W,), jnp.int32),
            pltpu.VMEM((_W,), jnp.int32),
            pltpu.VMEM((_W,), f32),
            pltpu.VMEM((_W,), f32),
            pltpu.VMEM((_W,), f32),
            pltpu.VMEM((_W,), f32),
            pltpu.VMEM((512,), jnp.int32),
            pltpu.VMEM((512,), jnp.int32),
            pltpu.VMEM((_C,), f32),
            pltpu.VMEM((_C,), f32),
            pltpu.VMEM((_C,), f32),
            pltpu.VMEM((_C,), f32),
            pltpu.VMEM((16,), jnp.int32),
        ],
        compiler_params=pltpu.CompilerParams(needs_layout_passes=False),
    )(keys, bx1, by1, bx2, by2)


def _make_nms_body(width, with_flag):
    def body(scores_ref, x1_ref, y1_ref, x2_ref, y2_ref, *refs):
        if with_flag:
            (cls_out, conf_out, ox1_out, oy1_out, ox2_out, oy2_out,
             flag_out, work_ref, area_ref) = refs
        else:
            (cls_out, conf_out, ox1_out, oy1_out, ox2_out, oy2_out,
             work_ref, area_ref) = refs

        scores = scores_ref[...]
        work_ref[...] = jnp.where(scores > _CONF_T, scores, -jnp.inf)
        x1 = x1_ref[...]
        y1 = y1_ref[...]
        x2 = x2_ref[...]
        y2 = y2_ref[...]
        area_ref[...] = (jnp.maximum(x2 - x1, 0.0)
                         * jnp.maximum(y2 - y1, 0.0))

        iota = lax.broadcasted_iota(jnp.int32, (_L, width), 1)
        lane = lax.broadcasted_iota(jnp.int32, (_L, 1), 0)
        clsvec = (lane % _NCLS + 1).astype(jnp.float32)
        col = lax.broadcasted_iota(jnp.int32, (_L, _KSEL), 1)

        def step(k, _):
            work = work_ref[...]
            m = jnp.max(work, axis=1, keepdims=True)
            msk = work == m
            idx = jnp.min(jnp.where(msk, iota, width), axis=1, keepdims=True)
            onehot = iota == idx

            x1 = x1_ref[...]
            y1 = y1_ref[...]
            x2 = x2_ref[...]
            y2 = y2_ref[...]
            zero = jnp.zeros_like(x1)
            sx1 = jnp.sum(jnp.where(onehot, x1, zero), axis=1, keepdims=True)
            sy1 = jnp.sum(jnp.where(onehot, y1, zero), axis=1, keepdims=True)
            sx2 = jnp.sum(jnp.where(onehot, x2, zero), axis=1, keepdims=True)
            sy2 = jnp.sum(jnp.where(onehot, y2, zero), axis=1, keepdims=True)

            xi1 = jnp.maximum(sx1, x1)
            yi1 = jnp.maximum(sy1, y1)
            xi2 = jnp.minimum(sx2, x2)
            yi2 = jnp.minimum(sy2, y2)
            inter = (jnp.maximum(xi2 - xi1, 0.0)
                     * jnp.maximum(yi2 - yi1, 0.0))
            a1 = (jnp.maximum(sx2 - sx1, 0.0)
                  * jnp.maximum(sy2 - sy1, 0.0))
            iou = inter / (a1 + area_ref[...] - inter + 1e-8)

            supp = (iou >= _IOU_T) | onehot
            work_ref[...] = jnp.where(supp, -jnp.inf, work)

            ok = m > 0.0
            z1 = jnp.zeros_like(m)
            here = col == k
            for ref, val in ((cls_out, clsvec), (conf_out, m),
                             (ox1_out, sx1), (oy1_out, sy1),
                             (ox2_out, sx2), (oy2_out, sy2)):
                v = jnp.where(ok, val, z1)
                ref[...] = jnp.where(here, v, ref[...])
            if with_flag:
                flag_out[...] = jnp.broadcast_to(
                    jnp.where(ok, z1, z1 + 1.0), (_L, 128))
            return 0

        lax.fori_loop(0, _KSEL, step, 0)

    return body


def _tc_nms(scores, x1, y1, x2, y2, width, with_flag):
    f32 = jnp.float32
    out_sd = [jax.ShapeDtypeStruct((_L, _KSEL), f32)] * 6
    if with_flag:
        out_sd = out_sd + [jax.ShapeDtypeStruct((_L, 128), f32)]
    return pl.pallas_call(
        _make_nms_body(width, with_flag),
        out_shape=out_sd,
        scratch_shapes=[
            pltpu.VMEM((_L, width), f32),
            pltpu.VMEM((_L, width), f32),
        ],
    )(scores, x1, y1, x2, y2)


def _decode_boxes(y_pred):
    cx = y_pred[..., -12] * y_pred[..., -4] * y_pred[..., -6] + y_pred[..., -8]
    cy = y_pred[..., -11] * y_pred[..., -3] * y_pred[..., -5] + y_pred[..., -7]
    w = jnp.exp(y_pred[..., -10] * y_pred[..., -2]) * y_pred[..., -6]
    h = jnp.exp(y_pred[..., -9] * y_pred[..., -1]) * y_pred[..., -5]
    xmin = (cx - 0.5 * w) * _IMG_W
    ymin = (cy - 0.5 * h) * _IMG_H
    xmax = (cx + 0.5 * w) * _IMG_W
    ymax = (cy + 0.5 * h) * _IMG_H
    return xmin, ymin, xmax, ymax


def kernel(y_pred):
    xmin, ymin, xmax, ymax = _decode_boxes(y_pred)
    confs = y_pred[..., 1:_NCLS + 1]

    pad = _W - _N
    scores = jnp.transpose(confs, (0, 2, 1))
    scores = jnp.pad(scores, ((0, 0), (0, 0), (0, pad))).reshape(_L, _W)
    sbits = lax.bitcast_convert_type(scores, jnp.int32)
    keys = jnp.where(sbits > _KTHR, _KMAX - sbits, _KINACT)

    boxes_b = [jnp.pad(a, ((0, 0), (0, pad)))
               for a in (xmin, ymin, xmax, ymax)]

    skey, sx1, sy1, sx2, sy2, cnt = _sc_sort(keys, *boxes_b)
    sscores = lax.bitcast_convert_type(_KMAX - skey, jnp.float32)

    fast = _tc_nms(sscores, sx1, sy1, sx2, sy2, _C, True)
    fast_rows, flag = fast[:6], fast[6]

    need_fb = jnp.any((flag[:, 0] > 0.0) & (cnt[:, 0] > _C))

    def fallback(_):
        def lanes(a):
            return jnp.broadcast_to(
                a[:, None, :], (_B, _NCLS, _W)).reshape(_L, _W)
        full = _tc_nms(scores, *[lanes(a) for a in boxes_b], _W, False)
        return tuple(full)

    rows = lax.cond(need_fb, fallback, lambda _: tuple(fast_rows), None)

    flat_conf = rows[1].reshape(_B, _NCLS * _KSEL)
    _, top_idx = lax.top_k(flat_conf, _TOPK)
    gathered = [jnp.take_along_axis(f.reshape(_B, _NCLS * _KSEL),
                                    top_idx, axis=1)
                for f in rows]
    return jnp.stack(gathered, axis=-1)


# speedup vs baseline: 74.9950x; 1.2792x over previous
import functools

import jax
import jax.numpy as jnp
from jax import lax
from jax.experimental import pallas as pl
from jax.experimental.pallas import tpu as pltpu
from jax.experimental.pallas import tpu_sc as plsc

_CONF_T = 0.01
_IOU_T = 0.45
_TOPK = 200
_KSEL = 200
_B = 4
_N = 8732
_NCLS = 20
_L = _B * _NCLS
_W = 8960
_C = 1024
_IMG_H = 300.0
_IMG_W = 300.0

_KMAX = 0x3F7FFFFF
_KTHR = 0x3C23D70A
_KINACT = _KMAX - _KTHR
_NV = _W // 16
_NVC = _C // 16
_NW = 32


def _sc_sort_body(keys_hbm, x1_hbm, y1_hbm, x2_hbm, y2_hbm,
                  okey, ox1, oy1, ox2, oy2, ocnt,
                  kA, iA, kB, iB, vx1, vy1, vx2, vy2,
                  hist, base, s1, s2, s3, s4, scnt):
    cid = lax.axis_index("c")
    sid = lax.axis_index("s")
    wid = sid * 2 + cid

    def process(lane):
        batch = ((lane >= _NCLS).astype(jnp.int32)
                 + (lane >= 2 * _NCLS) + (lane >= 3 * _NCLS))
        pltpu.sync_copy(keys_hbm.at[lane], kA)

        kthr = _KINACT
        lane16 = lax.iota(jnp.int32, 16)

        one16 = jnp.ones((16,), jnp.int32)
        zero16 = jnp.zeros((16,), jnp.int32)

        def ibody(i, acc):
            s = pl.ds(i * 16, 16)
            iA[s] = lane16 + i * 16
            return acc + jnp.sum(jnp.where(kA[s] < kthr, one16, zero16))

        acc = lax.fori_loop(0, _NV, ibody, jnp.int32(0))
        scnt[...] = jnp.broadcast_to(acc, (16,))
        pltpu.sync_copy(scnt, ocnt.at[lane])

        z16 = jnp.zeros((16,), jnp.int32)
        big16 = jnp.full((16,), jnp.int32(_W + 16))
        bufs = ((kA, iA, kB, iB), (kB, iB, kA, iA))

        def radix_pass(p, shift, nvr):
            src_k, src_i, dst_k, dst_i = bufs[p % 2]

            def zbody(j, _):
                hist[pl.ds(j * 16, 16)] = z16
                return 0

            lax.fori_loop(0, 32, zbody, 0)

            def hbody(i, _):
                dig = (src_k[pl.ds(i * 16, 16)] >> shift) & 511
                cnt, last = plsc.scan_count(dig)
                plsc.addupdate_scatter(hist, [dig], cnt, mask=last)
                return 0

            lax.fori_loop(0, nvr, hbody, 0)

            def sbody(j, carry):
                h = hist[pl.ds(j * 16, 16)]
                c = plsc.cumsum(h)
                base[pl.ds(j * 16, 16)] = c - h + carry
                return carry + jnp.max(c)

            lax.fori_loop(0, 32, sbody, jnp.int32(0))

            def pbody(i, _):
                s = pl.ds(i * 16, 16)
                kv = src_k[s]
                iv = src_i[s]
                dig = (kv >> shift) & 511
                cnt, last = plsc.scan_count(dig)
                pos = plsc.load_gather(base, [dig]) + cnt - 1
                plsc.store_scatter(dst_k, [pos], kv)
                plsc.store_scatter(dst_i, [pos], iv)
                plsc.addupdate_scatter(base, [dig], cnt, mask=last)
                return 0

            lax.fori_loop(0, nvr, pbody, 0)

        radix_pass(0, 18, _NV)

        def mbody(j, mv):
            e = base[pl.ds(j * 16, 16)]
            return jnp.minimum(mv, jnp.where(e >= _C, e, big16))

        mvec = lax.fori_loop(0, 32, mbody, big16)
        m16 = (jnp.min(mvec) + 15) >> 4

        radix_pass(1, 0, m16)
        radix_pass(2, 9, m16)
        radix_pass(3, 18, m16)

        pltpu.sync_copy(x1_hbm.at[batch], vx1)
        pltpu.sync_copy(y1_hbm.at[batch], vy1)
        pltpu.sync_copy(x2_hbm.at[batch], vx2)
        pltpu.sync_copy(y2_hbm.at[batch], vy2)

        def gbody(j, _):
            s = pl.ds(j * 16, 16)
            iv = iA[s]
            s1[s] = plsc.load_gather(vx1, [iv])
            s2[s] = plsc.load_gather(vy1, [iv])
            s3[s] = plsc.load_gather(vx2, [iv])
            s4[s] = plsc.load_gather(vy2, [iv])
            return 0

        lax.fori_loop(0, _NVC, gbody, 0)

        pltpu.sync_copy(kA.at[pl.ds(0, _C)], okey.at[lane])
        pltpu.sync_copy(s1, ox1.at[lane])
        pltpu.sync_copy(s2, oy1.at[lane])
        pltpu.sync_copy(s3, ox2.at[lane])
        pltpu.sync_copy(s4, oy2.at[lane])

    for r in range(3):
        lane = wid + _NW * r
        if r < 2:
            process(lane)
        else:
            @pl.when(lane < _L)
            def _():
                process(lane)


@functools.partial(jax.jit, static_argnums=())
def _sc_sort(keys, bx1, by1, bx2, by2):
    mesh = plsc.VectorSubcoreMesh(core_axis_name="c", subcore_axis_name="s",
                                  num_cores=2, num_subcores=16)
    f32 = jnp.float32
    return pl.kernel(
        _sc_sort_body,
        out_type=[
            jax.ShapeDtypeStruct((_L, _C), jnp.int32),
            jax.ShapeDtypeStruct((_L, _C), f32),
            jax.ShapeDtypeStruct((_L, _C), f32),
            jax.ShapeDtypeStruct((_L, _C), f32),
            jax.ShapeDtypeStruct((_L, _C), f32),
            jax.ShapeDtypeStruct((_L, 16), jnp.int32),
        ],
        mesh=mesh,
        scratch_types=[
            pltpu.VMEM((_W,), jnp.int32),
            pltpu.VMEM((_W,), jnp.int32),
            pltpu.VMEM((_W,), jnp.int32),
            pltpu.VMEM((_W,), jnp.int32),
            pltpu.VMEM((_W,), f32),
            pltpu.VMEM((_W,), f32),
            pltpu.VMEM((_W,), f32),
            pltpu.VMEM((_W,), f32),
            pltpu.VMEM((512,), jnp.int32),
            pltpu.VMEM((512,), jnp.int32),
            pltpu.VMEM((_C,), f32),
            pltpu.VMEM((_C,), f32),
            pltpu.VMEM((_C,), f32),
            pltpu.VMEM((_C,), f32),
            pltpu.VMEM((16,), jnp.int32),
        ],
        compiler_params=pltpu.CompilerParams(needs_layout_passes=False),
    )(keys, bx1, by1, bx2, by2)


def _make_nms_body(width, with_flag):
    def body(scores_ref, x1_ref, y1_ref, x2_ref, y2_ref, *refs):
        if with_flag:
            (cls_out, conf_out, ox1_out, oy1_out, ox2_out, oy2_out,
             flag_out, work_ref, area_ref) = refs
        else:
            (cls_out, conf_out, ox1_out, oy1_out, ox2_out, oy2_out,
             work_ref, area_ref) = refs

        scores = scores_ref[...]
        work_ref[...] = jnp.where(scores > _CONF_T, scores, -jnp.inf)
        x1 = x1_ref[...]
        y1 = y1_ref[...]
        x2 = x2_ref[...]
        y2 = y2_ref[...]
        area_ref[...] = (jnp.maximum(x2 - x1, 0.0)
                         * jnp.maximum(y2 - y1, 0.0))

        iota = lax.broadcasted_iota(jnp.int32, (_L, width), 1)
        lane = lax.broadcasted_iota(jnp.int32, (_L, 1), 0)
        clsvec = (lane % _NCLS + 1).astype(jnp.float32)
        col = lax.broadcasted_iota(jnp.int32, (_L, _KSEL), 1)

        def step(k, _):
            work = work_ref[...]
            m = jnp.max(work, axis=1, keepdims=True)
            msk = work == m
            idx = jnp.min(jnp.where(msk, iota, width), axis=1, keepdims=True)
            onehot = iota == idx

            x1 = x1_ref[...]
            y1 = y1_ref[...]
            x2 = x2_ref[...]
            y2 = y2_ref[...]
            zero = jnp.zeros_like(x1)
            sx1 = jnp.sum(jnp.where(onehot, x1, zero), axis=1, keepdims=True)
            sy1 = jnp.sum(jnp.where(onehot, y1, zero), axis=1, keepdims=True)
            sx2 = jnp.sum(jnp.where(onehot, x2, zero), axis=1, keepdims=True)
            sy2 = jnp.sum(jnp.where(onehot, y2, zero), axis=1, keepdims=True)

            xi1 = jnp.maximum(sx1, x1)
            yi1 = jnp.maximum(sy1, y1)
            xi2 = jnp.minimum(sx2, x2)
            yi2 = jnp.minimum(sy2, y2)
            inter = (jnp.maximum(xi2 - xi1, 0.0)
                     * jnp.maximum(yi2 - yi1, 0.0))
            a1 = (jnp.maximum(sx2 - sx1, 0.0)
                  * jnp.maximum(sy2 - sy1, 0.0))
            iou = inter / (a1 + area_ref[...] - inter + 1e-8)

            supp = (iou >= _IOU_T) | onehot
            work_ref[...] = jnp.where(supp, -jnp.inf, work)

            ok = m > 0.0
            z1 = jnp.zeros_like(m)
            here = col == k
            for ref, val in ((cls_out, clsvec), (conf_out, m),
                             (ox1_out, sx1), (oy1_out, sy1),
                             (ox2_out, sx2), (oy2_out, sy2)):
                v = jnp.where(ok, val, z1)
                ref[...] = jnp.where(here, v, ref[...])
            if with_flag:
                flag_out[...] = jnp.broadcast_to(
                    jnp.where(ok, z1, z1 + 1.0), (_L, 128))
            return 0

        lax.fori_loop(0, _KSEL, step, 0)

    return body


def _tc_nms(scores, x1, y1, x2, y2, width, with_flag):
    f32 = jnp.float32
    out_sd = [jax.ShapeDtypeStruct((_L, _KSEL), f32)] * 6
    if with_flag:
        out_sd = out_sd + [jax.ShapeDtypeStruct((_L, 128), f32)]
    return pl.pallas_call(
        _make_nms_body(width, with_flag),
        out_shape=out_sd,
        scratch_shapes=[
            pltpu.VMEM((_L, width), f32),
            pltpu.VMEM((_L, width), f32),
        ],
    )(scores, x1, y1, x2, y2)


def _decode_boxes(y_pred):
    cx = y_pred[..., -12] * y_pred[..., -4] * y_pred[..., -6] + y_pred[..., -8]
    cy = y_pred[..., -11] * y_pred[..., -3] * y_pred[..., -5] + y_pred[..., -7]
    w = jnp.exp(y_pred[..., -10] * y_pred[..., -2]) * y_pred[..., -6]
    h = jnp.exp(y_pred[..., -9] * y_pred[..., -1]) * y_pred[..., -5]
    xmin = (cx - 0.5 * w) * _IMG_W
    ymin = (cy - 0.5 * h) * _IMG_H
    xmax = (cx + 0.5 * w) * _IMG_W
    ymax = (cy + 0.5 * h) * _IMG_H
    return xmin, ymin, xmax, ymax


def kernel(y_pred):
    xmin, ymin, xmax, ymax = _decode_boxes(y_pred)
    confs = y_pred[..., 1:_NCLS + 1]

    pad = _W - _N
    scores = jnp.transpose(confs, (0, 2, 1))
    scores = jnp.pad(scores, ((0, 0), (0, 0), (0, pad))).reshape(_L, _W)
    sbits = lax.bitcast_convert_type(scores, jnp.int32)
    keys = jnp.where(sbits > _KTHR, _KMAX - sbits, _KINACT)

    boxes_b = [jnp.pad(a, ((0, 0), (0, pad)))
               for a in (xmin, ymin, xmax, ymax)]

    skey, sx1, sy1, sx2, sy2, cnt = _sc_sort(keys, *boxes_b)
    sscores = lax.bitcast_convert_type(_KMAX - skey, jnp.float32)

    fast = _tc_nms(sscores, sx1, sy1, sx2, sy2, _C, True)
    fast_rows, flag = fast[:6], fast[6]

    need_fb = jnp.any((flag[:, 0] > 0.0) & (cnt[:, 0] > _C))

    def fallback(_):
        def lanes(a):
            return jnp.broadcast_to(
                a[:, None, :], (_B, _NCLS, _W)).reshape(_L, _W)
        full = _tc_nms(scores, *[lanes(a) for a in boxes_b], _W, False)
        return tuple(full)

    rows = lax.cond(need_fb, fallback, lambda _: tuple(fast_rows), None)

    flat_conf = rows[1].reshape(_B, _NCLS * _KSEL)
    _, top_idx = lax.top_k(flat_conf, _TOPK)
    gathered = [jnp.take_along_axis(f.reshape(_B, _NCLS * _KSEL),
                                    top_idx, axis=1)
                for f in rows]
    return jnp.stack(gathered, axis=-1)


# speedup vs baseline: 106.4285x; 1.4191x over previous
import functools

import jax
import jax.numpy as jnp
from jax import lax
from jax.experimental import pallas as pl
from jax.experimental.pallas import tpu as pltpu
from jax.experimental.pallas import tpu_sc as plsc

_CONF_T = 0.01
_IOU_T = 0.45
_TOPK = 200
_KSEL = 48
_KFULL = 200
_B = 4
_N = 8732
_NCLS = 20
_L = _B * _NCLS
_W = 8960
_C = 1024
_IMG_H = 300.0
_IMG_W = 300.0

_KMAX = 0x3F7FFFFF
_KTHR = 0x3C23D70A
_KINACT = _KMAX - _KTHR
_NV = _W // 16
_NVC = _C // 16
_NW = 32


def _sc_sort_body(keys_hbm, x1_hbm, y1_hbm, x2_hbm, y2_hbm,
                  okey, ox1, oy1, ox2, oy2, ocnt,
                  kA, iA, kB, iB, vx1, vy1, vx2, vy2,
                  hist, base, s1, s2, s3, s4, scnt):
    cid = lax.axis_index("c")
    sid = lax.axis_index("s")
    wid = sid * 2 + cid

    def process(lane):
        batch = ((lane >= _NCLS).astype(jnp.int32)
                 + (lane >= 2 * _NCLS) + (lane >= 3 * _NCLS))
        pltpu.sync_copy(keys_hbm.at[lane], kA)

        kthr = _KINACT
        lane16 = lax.iota(jnp.int32, 16)

        one16 = jnp.ones((16,), jnp.int32)
        zero16 = jnp.zeros((16,), jnp.int32)

        def ibody(i, acc):
            s = pl.ds(i * 16, 16)
            iA[s] = lane16 + i * 16
            return acc + jnp.sum(jnp.where(kA[s] < kthr, one16, zero16))

        acc = lax.fori_loop(0, _NV, ibody, jnp.int32(0))
        scnt[...] = jnp.broadcast_to(acc, (16,))
        pltpu.sync_copy(scnt, ocnt.at[lane])

        z16 = jnp.zeros((16,), jnp.int32)
        big16 = jnp.full((16,), jnp.int32(_W + 16))
        bufs = ((kA, iA, kB, iB), (kB, iB, kA, iA))

        def radix_pass(p, shift, nvr):
            src_k, src_i, dst_k, dst_i = bufs[p % 2]

            def zbody(j, _):
                hist[pl.ds(j * 16, 16)] = z16
                return 0

            lax.fori_loop(0, 32, zbody, 0)

            def hbody(i, _):
                dig = (src_k[pl.ds(i * 16, 16)] >> shift) & 511
                cnt, last = plsc.scan_count(dig)
                plsc.addupdate_scatter(hist, [dig], cnt, mask=last)
                return 0

            lax.fori_loop(0, nvr, hbody, 0)

            def sbody(j, carry):
                h = hist[pl.ds(j * 16, 16)]
                c = plsc.cumsum(h)
                base[pl.ds(j * 16, 16)] = c - h + carry
                return carry + jnp.max(c)

            lax.fori_loop(0, 32, sbody, jnp.int32(0))

            def pbody(i, _):
                s = pl.ds(i * 16, 16)
                kv = src_k[s]
                iv = src_i[s]
                dig = (kv >> shift) & 511
                cnt, last = plsc.scan_count(dig)
                pos = plsc.load_gather(base, [dig]) + cnt - 1
                plsc.store_scatter(dst_k, [pos], kv)
                plsc.store_scatter(dst_i, [pos], iv)
                plsc.addupdate_scatter(base, [dig], cnt, mask=last)
                return 0

            lax.fori_loop(0, nvr, pbody, 0)

        radix_pass(0, 18, _NV)

        def mbody(j, mv):
            e = base[pl.ds(j * 16, 16)]
            return jnp.minimum(mv, jnp.where(e >= _C, e, big16))

        mvec = lax.fori_loop(0, 32, mbody, big16)
        m16 = (jnp.min(mvec) + 15) >> 4

        radix_pass(1, 0, m16)
        radix_pass(2, 9, m16)
        radix_pass(3, 18, m16)

        pltpu.sync_copy(x1_hbm.at[batch], vx1)
        pltpu.sync_copy(y1_hbm.at[batch], vy1)
        pltpu.sync_copy(x2_hbm.at[batch], vx2)
        pltpu.sync_copy(y2_hbm.at[batch], vy2)

        def gbody(j, _):
            s = pl.ds(j * 16, 16)
            iv = iA[s]
            s1[s] = plsc.load_gather(vx1, [iv])
            s2[s] = plsc.load_gather(vy1, [iv])
            s3[s] = plsc.load_gather(vx2, [iv])
            s4[s] = plsc.load_gather(vy2, [iv])
            return 0

        lax.fori_loop(0, _NVC, gbody, 0)

        pltpu.sync_copy(kA.at[pl.ds(0, _C)], okey.at[lane])
        pltpu.sync_copy(s1, ox1.at[lane])
        pltpu.sync_copy(s2, oy1.at[lane])
        pltpu.sync_copy(s3, ox2.at[lane])
        pltpu.sync_copy(s4, oy2.at[lane])

    for r in range(3):
        lane = wid + _NW * r
        if r < 2:
            process(lane)
        else:
            @pl.when(lane < _L)
            def _():
                process(lane)


@functools.partial(jax.jit, static_argnums=())
def _sc_sort(keys, bx1, by1, bx2, by2):
    mesh = plsc.VectorSubcoreMesh(core_axis_name="c", subcore_axis_name="s",
                                  num_cores=2, num_subcores=16)
    f32 = jnp.float32
    return pl.kernel(
        _sc_sort_body,
        out_type=[
            jax.ShapeDtypeStruct((_L, _C), jnp.int32),
            jax.ShapeDtypeStruct((_L, _C), f32),
            jax.ShapeDtypeStruct((_L, _C), f32),
            jax.ShapeDtypeStruct((_L, _C), f32),
            jax.ShapeDtypeStruct((_L, _C), f32),
            jax.ShapeDtypeStruct((_L, 16), jnp.int32),
        ],
        mesh=mesh,
        scratch_types=[
            pltpu.VMEM((_W,), jnp.int32),
            pltpu.VMEM((_W,), jnp.int32),
            pltpu.VMEM((_W,), jnp.int32),
            pltpu.VMEM((_W,), jnp.int32),
            pltpu.VMEM((_W,), f32),
            pltpu.VMEM((_W,), f32),
            pltpu.VMEM((_W,), f32),
            pltpu.VMEM((_W,), f32),
            pltpu.VMEM((512,), jnp.int32),
            pltpu.VMEM((512,), jnp.int32),
            pltpu.VMEM((_C,), f32),
            pltpu.VMEM((_C,), f32),
            pltpu.VMEM((_C,), f32),
            pltpu.VMEM((_C,), f32),
            pltpu.VMEM((16,), jnp.int32),
        ],
        compiler_params=pltpu.CompilerParams(needs_layout_passes=False),
    )(keys, bx1, by1, bx2, by2)


def _make_nms_body(width, ksel, with_flag):
    def body(scores_ref, x1_ref, y1_ref, x2_ref, y2_ref, *refs):
        if with_flag:
            (cls_out, conf_out, ox1_out, oy1_out, ox2_out, oy2_out,
             flag_out, work_ref, area_ref) = refs
        else:
            (cls_out, conf_out, ox1_out, oy1_out, ox2_out, oy2_out,
             work_ref, area_ref) = refs

        scores = scores_ref[...]
        work_ref[...] = jnp.where(scores > _CONF_T, scores, -jnp.inf)
        x1 = x1_ref[...]
        y1 = y1_ref[...]
        x2 = x2_ref[...]
        y2 = y2_ref[...]
        area_ref[...] = (jnp.maximum(x2 - x1, 0.0)
                         * jnp.maximum(y2 - y1, 0.0))

        iota = lax.broadcasted_iota(jnp.int32, (_L, width), 1)
        lane = lax.broadcasted_iota(jnp.int32, (_L, 1), 0)
        clsvec = (lane % _NCLS + 1).astype(jnp.float32)
        col = lax.broadcasted_iota(jnp.int32, (_L, ksel), 1)

        def step(k, _):
            work = work_ref[...]
            m = jnp.max(work, axis=1, keepdims=True)
            msk = work == m
            idx = jnp.min(jnp.where(msk, iota, width), axis=1, keepdims=True)
            onehot = iota == idx

            x1 = x1_ref[...]
            y1 = y1_ref[...]
            x2 = x2_ref[...]
            y2 = y2_ref[...]
            zero = jnp.zeros_like(x1)
            sx1 = jnp.sum(jnp.where(onehot, x1, zero), axis=1, keepdims=True)
            sy1 = jnp.sum(jnp.where(onehot, y1, zero), axis=1, keepdims=True)
            sx2 = jnp.sum(jnp.where(onehot, x2, zero), axis=1, keepdims=True)
            sy2 = jnp.sum(jnp.where(onehot, y2, zero), axis=1, keepdims=True)

            xi1 = jnp.maximum(sx1, x1)
            yi1 = jnp.maximum(sy1, y1)
            xi2 = jnp.minimum(sx2, x2)
            yi2 = jnp.minimum(sy2, y2)
            inter = (jnp.maximum(xi2 - xi1, 0.0)
                     * jnp.maximum(yi2 - yi1, 0.0))
            a1 = (jnp.maximum(sx2 - sx1, 0.0)
                  * jnp.maximum(sy2 - sy1, 0.0))
            iou = inter / (a1 + area_ref[...] - inter + 1e-8)

            supp = (iou >= _IOU_T) | onehot
            work_ref[...] = jnp.where(supp, -jnp.inf, work)

            ok = m > 0.0
            z1 = jnp.zeros_like(m)
            here = col == k
            for ref, val in ((cls_out, clsvec), (conf_out, m),
                             (ox1_out, sx1), (oy1_out, sy1),
                             (ox2_out, sx2), (oy2_out, sy2)):
                v = jnp.where(ok, val, z1)
                ref[...] = jnp.where(here, v, ref[...])
            if with_flag:
                flag_out[...] = jnp.broadcast_to(
                    jnp.where(ok, z1, z1 + 1.0), (_L, 128))
            return 0

        lax.fori_loop(0, ksel, step, 0)

    return body


def _tc_nms(scores, x1, y1, x2, y2, width, ksel, with_flag):
    f32 = jnp.float32
    out_sd = [jax.ShapeDtypeStruct((_L, ksel), f32)] * 6
    if with_flag:
        out_sd = out_sd + [jax.ShapeDtypeStruct((_L, 128), f32)]
    return pl.pallas_call(
        _make_nms_body(width, ksel, with_flag),
        out_shape=out_sd,
        scratch_shapes=[
            pltpu.VMEM((_L, width), f32),
            pltpu.VMEM((_L, width), f32),
        ],
    )(scores, x1, y1, x2, y2)


def _decode_boxes(y_pred):
    cx = y_pred[..., -12] * y_pred[..., -4] * y_pred[..., -6] + y_pred[..., -8]
    cy = y_pred[..., -11] * y_pred[..., -3] * y_pred[..., -5] + y_pred[..., -7]
    w = jnp.exp(y_pred[..., -10] * y_pred[..., -2]) * y_pred[..., -6]
    h = jnp.exp(y_pred[..., -9] * y_pred[..., -1]) * y_pred[..., -5]
    xmin = (cx - 0.5 * w) * _IMG_W
    ymin = (cy - 0.5 * h) * _IMG_H
    xmax = (cx + 0.5 * w) * _IMG_W
    ymax = (cy + 0.5 * h) * _IMG_H
    return xmin, ymin, xmax, ymax


def kernel(y_pred):
    xmin, ymin, xmax, ymax = _decode_boxes(y_pred)
    confs = y_pred[..., 1:_NCLS + 1]

    pad = _W - _N
    scores = jnp.transpose(confs, (0, 2, 1))
    scores = jnp.pad(scores, ((0, 0), (0, 0), (0, pad))).reshape(_L, _W)
    sbits = lax.bitcast_convert_type(scores, jnp.int32)
    keys = jnp.where(sbits > _KTHR, _KMAX - sbits, _KINACT)

    boxes_b = [jnp.pad(a, ((0, 0), (0, pad)))
               for a in (xmin, ymin, xmax, ymax)]

    skey, sx1, sy1, sx2, sy2, cnt = _sc_sort(keys, *boxes_b)
    sscores = lax.bitcast_convert_type(_KMAX - skey, jnp.float32)

    fast = _tc_nms(sscores, sx1, sy1, sx2, sy2, _C, _KSEL, True)
    fast_rows, flag = fast[:6], fast[6]
    exhausted = flag[:, 0] > 0.0

    flat_conf = fast_rows[1].reshape(_B, _NCLS * _KSEL)
    _, top_idx = lax.top_k(flat_conf, _TOPK)
    v200 = jnp.take_along_axis(flat_conf, top_idx[:, -1:], axis=1)
    v_lane = jnp.repeat(v200[:, 0], _NCLS)

    conf_last = fast_rows[1][:, _KSEL - 1]
    bad_a = (~exhausted) & (conf_last >= v_lane)
    bad_b = exhausted & (cnt[:, 0] > _C)
    need_fb = jnp.any(bad_a | bad_b)

    def fast_path(_):
        gathered = [jnp.take_along_axis(f.reshape(_B, _NCLS * _KSEL),
                                        top_idx, axis=1)
                    for f in fast_rows]
        return jnp.stack(gathered, axis=-1)

    def fallback(_):
        def lanes(a):
            return jnp.broadcast_to(
                a[:, None, :], (_B, _NCLS, _W)).reshape(_L, _W)
        full = _tc_nms(scores, *[lanes(a) for a in boxes_b], _W, _KFULL,
                       False)
        fc = full[1].reshape(_B, _NCLS * _KFULL)
        _, ti = lax.top_k(fc, _TOPK)
        gathered = [jnp.take_along_axis(f.reshape(_B, _NCLS * _KFULL),
                                        ti, axis=1)
                    for f in full]
        return jnp.stack(gathered, axis=-1)

    return lax.cond(need_fb, fallback, fast_path, None)


# speedup vs baseline: 128.0561x; 1.2032x over previous
import functools

import jax
import jax.numpy as jnp
from jax import lax
from jax.experimental import pallas as pl
from jax.experimental.pallas import tpu as pltpu
from jax.experimental.pallas import tpu_sc as plsc

_CONF_T = 0.01
_IOU_T = 0.45
_TOPK = 200
_KSEL = 48
_KFULL = 200
_B = 4
_N = 8732
_NCLS = 20
_L = _B * _NCLS
_W = 8960
_C = 1024
_IMG_H = 300.0
_IMG_W = 300.0

_KMAX = 0x3F7FFFFF
_KTHR = 0x3C23D70A
_KINACT = _KMAX - _KTHR
_NV = _W // 16
_NVC = _C // 16
_NW = 32


def _sc_sort_body(keys_hbm, x1_hbm, y1_hbm, x2_hbm, y2_hbm,
                  okey, ox1, oy1, ox2, oy2, ocnt,
                  kA, iA, kB, iB, vx1, vy1, vx2, vy2,
                  hist, base, s1, s2, s3, s4, scnt):
    cid = lax.axis_index("c")
    sid = lax.axis_index("s")
    wid = sid * 2 + cid

    def process(lane):
        batch = ((lane >= _NCLS).astype(jnp.int32)
                 + (lane >= 2 * _NCLS) + (lane >= 3 * _NCLS))
        pltpu.sync_copy(keys_hbm.at[lane], kA)

        kthr = _KINACT
        lane16 = lax.iota(jnp.int32, 16)

        one16 = jnp.ones((16,), jnp.int32)
        zero16 = jnp.zeros((16,), jnp.int32)

        z16 = jnp.zeros((16,), jnp.int32)
        big16 = jnp.full((16,), jnp.int32(_W + 16))
        bufs = ((kA, iA, kB, iB), (kB, iB, kA, iA))

        def radix_pass(p, shift, nvr, count_active=False, limit=None):
            src_k, src_i, dst_k, dst_i = bufs[p % 2]

            def zbody(j, _):
                hist[pl.ds(j * 16, 16)] = z16
                return 0

            lax.fori_loop(0, 32, zbody, 0)

            def hbody(i, acc):
                kv = src_k[pl.ds(i * 16, 16)]
                dig = (kv >> shift) & 511
                cnt, last = plsc.scan_count(dig)
                plsc.addupdate_scatter(hist, [dig], cnt, mask=last)
                if count_active:
                    acc = acc + jnp.sum(jnp.where(kv < kthr, one16, zero16))
                return acc

            acc = lax.fori_loop(0, nvr, hbody, jnp.int32(0))
            if count_active:
                scnt[...] = jnp.broadcast_to(acc, (16,))
                pltpu.sync_copy(scnt, ocnt.at[lane])

            def sbody(j, carry):
                h = hist[pl.ds(j * 16, 16)]
                c = plsc.cumsum(h)
                base[pl.ds(j * 16, 16)] = c - h + carry
                return carry + jnp.max(c)

            lax.fori_loop(0, 32, sbody, jnp.int32(0))

            limit = None
            m16 = None
            if p == 0:
                def mbody(j, mv):
                    sl = pl.ds(j * 16, 16)
                    e = base[sl] + hist[sl]
                    return jnp.minimum(mv, jnp.where(e >= _C, e, big16))

                mvec = lax.fori_loop(0, 32, mbody, big16)
                m16 = (jnp.min(mvec) + 15) >> 4
                limit = m16 * 16

            def pbody(i, _):
                s = pl.ds(i * 16, 16)
                kv = src_k[s]
                if p == 0:
                    iv = lane16 + i * 16
                else:
                    iv = src_i[s]
                dig = (kv >> shift) & 511
                cnt, last = plsc.scan_count(dig)
                pos = plsc.load_gather(base, [dig]) + cnt - 1
                if limit is None:
                    plsc.store_scatter(dst_k, [pos], kv)
                    plsc.store_scatter(dst_i, [pos], iv)
                else:
                    inreg = pos < limit
                    plsc.store_scatter(dst_k, [pos], kv, mask=inreg)
                    plsc.store_scatter(dst_i, [pos], iv, mask=inreg)
                plsc.addupdate_scatter(base, [dig], cnt, mask=last)
                return 0

            lax.fori_loop(0, nvr, pbody, 0)
            return m16

        m16 = radix_pass(0, 18, _NV, count_active=True)
        radix_pass(1, 0, m16)
        radix_pass(2, 9, m16)
        radix_pass(3, 18, m16)

        pltpu.sync_copy(x1_hbm.at[batch], vx1)
        pltpu.sync_copy(y1_hbm.at[batch], vy1)
        pltpu.sync_copy(x2_hbm.at[batch], vx2)
        pltpu.sync_copy(y2_hbm.at[batch], vy2)

        def gbody(j, _):
            s = pl.ds(j * 16, 16)
            iv = iA[s]
            s1[s] = plsc.load_gather(vx1, [iv])
            s2[s] = plsc.load_gather(vy1, [iv])
            s3[s] = plsc.load_gather(vx2, [iv])
            s4[s] = plsc.load_gather(vy2, [iv])
            return 0

        lax.fori_loop(0, _NVC, gbody, 0)

        pltpu.sync_copy(kA.at[pl.ds(0, _C)], okey.at[lane])
        pltpu.sync_copy(s1, ox1.at[lane])
        pltpu.sync_copy(s2, oy1.at[lane])
        pltpu.sync_copy(s3, ox2.at[lane])
        pltpu.sync_copy(s4, oy2.at[lane])

    for r in range(3):
        lane = wid + _NW * r
        if r < 2:
            process(lane)
        else:
            @pl.when(lane < _L)
            def _():
                process(lane)


@functools.partial(jax.jit, static_argnums=())
def _sc_sort(keys, bx1, by1, bx2, by2):
    mesh = plsc.VectorSubcoreMesh(core_axis_name="c", subcore_axis_name="s",
                                  num_cores=2, num_subcores=16)
    f32 = jnp.float32
    return pl.kernel(
        _sc_sort_body,
        out_type=[
            jax.ShapeDtypeStruct((_L, _C), jnp.int32),
            jax.ShapeDtypeStruct((_L, _C), f32),
            jax.ShapeDtypeStruct((_L, _C), f32),
            jax.ShapeDtypeStruct((_L, _C), f32),
            jax.ShapeDtypeStruct((_L, _C), f32),
            jax.ShapeDtypeStruct((_L, 16), jnp.int32),
        ],
        mesh=mesh,
        scratch_types=[
            pltpu.VMEM((_W,), jnp.int32),
            pltpu.VMEM((_W,), jnp.int32),
            pltpu.VMEM((_W,), jnp.int32),
            pltpu.VMEM((_W,), jnp.int32),
            pltpu.VMEM((_W,), f32),
            pltpu.VMEM((_W,), f32),
            pltpu.VMEM((_W,), f32),
            pltpu.VMEM((_W,), f32),
            pltpu.VMEM((512,), jnp.int32),
            pltpu.VMEM((512,), jnp.int32),
            pltpu.VMEM((_C,), f32),
            pltpu.VMEM((_C,), f32),
            pltpu.VMEM((_C,), f32),
            pltpu.VMEM((_C,), f32),
            pltpu.VMEM((16,), jnp.int32),
        ],
        compiler_params=pltpu.CompilerParams(needs_layout_passes=False),
    )(keys, bx1, by1, bx2, by2)


def _make_nms_body(width, ksel, with_flag):
    def body(scores_ref, x1_ref, y1_ref, x2_ref, y2_ref, *refs):
        if with_flag:
            (cls_out, conf_out, ox1_out, oy1_out, ox2_out, oy2_out,
             flag_out, work_ref, area_ref) = refs
        else:
            (cls_out, conf_out, ox1_out, oy1_out, ox2_out, oy2_out,
             work_ref, area_ref) = refs

        scores = scores_ref[...]
        work_ref[...] = jnp.where(scores > _CONF_T, scores, -jnp.inf)
        x1 = x1_ref[...]
        y1 = y1_ref[...]
        x2 = x2_ref[...]
        y2 = y2_ref[...]
        area_ref[...] = (jnp.maximum(x2 - x1, 0.0)
                         * jnp.maximum(y2 - y1, 0.0))

        iota = lax.broadcasted_iota(jnp.int32, (_L, width), 1)
        lane = lax.broadcasted_iota(jnp.int32, (_L, 1), 0)
        clsvec = (lane % _NCLS + 1).astype(jnp.float32)
        col = lax.broadcasted_iota(jnp.int32, (_L, ksel), 1)

        def step(k, _):
            work = work_ref[...]
            m = jnp.max(work, axis=1, keepdims=True)
            msk = work == m
            idx = jnp.min(jnp.where(msk, iota, width), axis=1, keepdims=True)
            onehot = iota == idx

            x1 = x1_ref[...]
            y1 = y1_ref[...]
            x2 = x2_ref[...]
            y2 = y2_ref[...]
            zero = jnp.zeros_like(x1)
            sx1 = jnp.sum(jnp.where(onehot, x1, zero), axis=1, keepdims=True)
            sy1 = jnp.sum(jnp.where(onehot, y1, zero), axis=1, keepdims=True)
            sx2 = jnp.sum(jnp.where(onehot, x2, zero), axis=1, keepdims=True)
            sy2 = jnp.sum(jnp.where(onehot, y2, zero), axis=1, keepdims=True)

            xi1 = jnp.maximum(sx1, x1)
            yi1 = jnp.maximum(sy1, y1)
            xi2 = jnp.minimum(sx2, x2)
            yi2 = jnp.minimum(sy2, y2)
            inter = (jnp.maximum(xi2 - xi1, 0.0)
                     * jnp.maximum(yi2 - yi1, 0.0))
            a1 = (jnp.maximum(sx2 - sx1, 0.0)
                  * jnp.maximum(sy2 - sy1, 0.0))
            iou = inter / (a1 + area_ref[...] - inter + 1e-8)

            supp = (iou >= _IOU_T) | onehot
            work_ref[...] = jnp.where(supp, -jnp.inf, work)

            ok = m > 0.0
            z1 = jnp.zeros_like(m)
            here = col == k
            for ref, val in ((cls_out, clsvec), (conf_out, m),
                             (ox1_out, sx1), (oy1_out, sy1),
                             (ox2_out, sx2), (oy2_out, sy2)):
                v = jnp.where(ok, val, z1)
                ref[...] = jnp.where(here, v, ref[...])
            if with_flag:
                flag_out[...] = jnp.broadcast_to(
                    jnp.where(ok, z1, z1 + 1.0), (_L, 128))
            return 0

        lax.fori_loop(0, ksel, step, 0)

    return body


def _tc_nms(scores, x1, y1, x2, y2, width, ksel, with_flag):
    f32 = jnp.float32
    out_sd = [jax.ShapeDtypeStruct((_L, ksel), f32)] * 6
    if with_flag:
        out_sd = out_sd + [jax.ShapeDtypeStruct((_L, 128), f32)]
    return pl.pallas_call(
        _make_nms_body(width, ksel, with_flag),
        out_shape=out_sd,
        scratch_shapes=[
            pltpu.VMEM((_L, width), f32),
            pltpu.VMEM((_L, width), f32),
        ],
    )(scores, x1, y1, x2, y2)


def _decode_boxes(y_pred):
    cx = y_pred[..., -12] * y_pred[..., -4] * y_pred[..., -6] + y_pred[..., -8]
    cy = y_pred[..., -11] * y_pred[..., -3] * y_pred[..., -5] + y_pred[..., -7]
    w = jnp.exp(y_pred[..., -10] * y_pred[..., -2]) * y_pred[..., -6]
    h = jnp.exp(y_pred[..., -9] * y_pred[..., -1]) * y_pred[..., -5]
    xmin = (cx - 0.5 * w) * _IMG_W
    ymin = (cy - 0.5 * h) * _IMG_H
    xmax = (cx + 0.5 * w) * _IMG_W
    ymax = (cy + 0.5 * h) * _IMG_H
    return xmin, ymin, xmax, ymax


def kernel(y_pred):
    xmin, ymin, xmax, ymax = _decode_boxes(y_pred)
    confs = y_pred[..., 1:_NCLS + 1]

    pad = _W - _N
    scores = jnp.transpose(confs, (0, 2, 1))
    scores = jnp.pad(scores, ((0, 0), (0, 0), (0, pad))).reshape(_L, _W)
    sbits = lax.bitcast_convert_type(scores, jnp.int32)
    keys = jnp.where(sbits > _KTHR, _KMAX - sbits, _KINACT)

    boxes_b = [jnp.pad(a, ((0, 0), (0, pad)))
               for a in (xmin, ymin, xmax, ymax)]

    skey, sx1, sy1, sx2, sy2, cnt = _sc_sort(keys, *boxes_b)
    sscores = lax.bitcast_convert_type(_KMAX - skey, jnp.float32)

    fast = _tc_nms(sscores, sx1, sy1, sx2, sy2, _C, _KSEL, True)
    fast_rows, flag = fast[:6], fast[6]
    exhausted = flag[:, 0] > 0.0

    flat_conf = fast_rows[1].reshape(_B, _NCLS * _KSEL)
    _, top_idx = lax.top_k(flat_conf, _TOPK)
    v200 = jnp.take_along_axis(flat_conf, top_idx[:, -1:], axis=1)
    v_lane = jnp.repeat(v200[:, 0], _NCLS)

    conf_last = fast_rows[1][:, _KSEL - 1]
    bad_a = (~exhausted) & (conf_last >= v_lane)
    bad_b = exhausted & (cnt[:, 0] > _C)
    need_fb = jnp.any(bad_a | bad_b)

    def fast_path(_):
        stacked = jnp.stack([f.reshape(_B, _NCLS * _KSEL)
                             for f in fast_rows], axis=-1)
        return jnp.take_along_axis(stacked, top_idx[..., None], axis=1)

    def fallback(_):
        def lanes(a):
            return jnp.broadcast_to(
                a[:, None, :], (_B, _NCLS, _W)).reshape(_L, _W)
        full = _tc_nms(scores, *[lanes(a) for a in boxes_b], _W, _KFULL,
                       False)
        fc = full[1].reshape(_B, _NCLS * _KFULL)
        _, ti = lax.top_k(fc, _TOPK)
        stacked = jnp.stack([f.reshape(_B, _NCLS * _KFULL)
                             for f in full], axis=-1)
        return jnp.take_along_axis(stacked, ti[..., None], axis=1)

    return lax.cond(need_fb, fallback, fast_path, None)


# speedup vs baseline: 149.8369x; 1.1701x over previous
import functools

import jax
import jax.numpy as jnp
from jax import lax
from jax.experimental import pallas as pl
from jax.experimental.pallas import tpu as pltpu
from jax.experimental.pallas import tpu_sc as plsc

_CONF_T = 0.01
_IOU_T = 0.45
_TOPK = 200
_KSEL = 48
_KFULL = 200
_B = 4
_N = 8732
_NCLS = 20
_L = _B * _NCLS
_W = 8960
_C = 256
_IMG_H = 300.0
_IMG_W = 300.0

_KMAX = 0x3F7FFFFF
_KTHR = 0x3C23D70A
_KINACT = _KMAX - _KTHR
_NV = _W // 16
_NVC = _C // 16
_NW = 32


def _sc_sort_body(keys_hbm, x1_hbm, y1_hbm, x2_hbm, y2_hbm,
                  okey, ox1, oy1, ox2, oy2, ocnt,
                  kA, iA, kB, iB, vx1, vy1, vx2, vy2,
                  hist, base, s1, s2, s3, s4, scnt):
    cid = lax.axis_index("c")
    sid = lax.axis_index("s")
    wid = sid * 2 + cid

    def process(lane):
        batch = ((lane >= _NCLS).astype(jnp.int32)
                 + (lane >= 2 * _NCLS) + (lane >= 3 * _NCLS))
        pltpu.sync_copy(keys_hbm.at[lane], kA)

        kthr = _KINACT
        lane16 = lax.iota(jnp.int32, 16)

        one16 = jnp.ones((16,), jnp.int32)
        zero16 = jnp.zeros((16,), jnp.int32)

        z16 = jnp.zeros((16,), jnp.int32)
        big16 = jnp.full((16,), jnp.int32(_W + 16))
        bufs = ((kA, iA, kB, iB), (kB, iB, kA, iA))

        def radix_pass(p, shift, nvr, count_active=False, limit=None):
            src_k, src_i, dst_k, dst_i = bufs[p % 2]

            def zbody(j, _):
                hist[pl.ds(j * 16, 16)] = z16
                return 0

            lax.fori_loop(0, 32, zbody, 0)

            def hbody(i, acc):
                kv = src_k[pl.ds(i * 16, 16)]
                dig = (kv >> shift) & 511
                cnt, last = plsc.scan_count(dig)
                plsc.addupdate_scatter(hist, [dig], cnt, mask=last)
                if count_active:
                    acc = acc + jnp.sum(jnp.where(kv < kthr, one16, zero16))
                return acc

            acc = lax.fori_loop(0, nvr, hbody, jnp.int32(0))
            if count_active:
                scnt[...] = jnp.broadcast_to(acc, (16,))
                pltpu.sync_copy(scnt, ocnt.at[lane])

            def sbody(j, carry):
                h = hist[pl.ds(j * 16, 16)]
                c = plsc.cumsum(h)
                base[pl.ds(j * 16, 16)] = c - h + carry
                return carry + jnp.max(c)

            lax.fori_loop(0, 32, sbody, jnp.int32(0))

            limit = None
            m16 = None
            if p == 0:
                def mbody(j, mv):
                    sl = pl.ds(j * 16, 16)
                    e = base[sl] + hist[sl]
                    return jnp.minimum(mv, jnp.where(e >= _C, e, big16))

                mvec = lax.fori_loop(0, 32, mbody, big16)
                m16 = (jnp.min(mvec) + 15) >> 4
                limit = m16 * 16

            def pbody(i, _):
                s = pl.ds(i * 16, 16)
                kv = src_k[s]
                if p == 0:
                    iv = lane16 + i * 16
                else:
                    iv = src_i[s]
                dig = (kv >> shift) & 511
                cnt, last = plsc.scan_count(dig)
                pos = plsc.load_gather(base, [dig]) + cnt - 1
                if limit is None:
                    plsc.store_scatter(dst_k, [pos], kv)
                    plsc.store_scatter(dst_i, [pos], iv)
                else:
                    inreg = pos < limit
                    plsc.store_scatter(dst_k, [pos], kv, mask=inreg)
                    plsc.store_scatter(dst_i, [pos], iv, mask=inreg)
                plsc.addupdate_scatter(base, [dig], cnt, mask=last)
                return 0

            lax.fori_loop(0, nvr, pbody, 0)
            return m16

        m16 = radix_pass(0, 18, _NV, count_active=True)
        radix_pass(1, 0, m16)
        radix_pass(2, 9, m16)
        radix_pass(3, 18, m16)

        pltpu.sync_copy(x1_hbm.at[batch], vx1)
        pltpu.sync_copy(y1_hbm.at[batch], vy1)
        pltpu.sync_copy(x2_hbm.at[batch], vx2)
        pltpu.sync_copy(y2_hbm.at[batch], vy2)

        def gbody(j, _):
            s = pl.ds(j * 16, 16)
            iv = iA[s]
            s1[s] = plsc.load_gather(vx1, [iv])
            s2[s] = plsc.load_gather(vy1, [iv])
            s3[s] = plsc.load_gather(vx2, [iv])
            s4[s] = plsc.load_gather(vy2, [iv])
            return 0

        lax.fori_loop(0, _NVC, gbody, 0)

        pltpu.sync_copy(kA.at[pl.ds(0, _C)], okey.at[lane])
        pltpu.sync_copy(s1, ox1.at[lane])
        pltpu.sync_copy(s2, oy1.at[lane])
        pltpu.sync_copy(s3, ox2.at[lane])
        pltpu.sync_copy(s4, oy2.at[lane])

    for r in range(3):
        lane = wid + _NW * r
        if r < 2:
            process(lane)
        else:
            @pl.when(lane < _L)
            def _():
                process(lane)


@functools.partial(jax.jit, static_argnums=())
def _sc_sort(keys, bx1, by1, bx2, by2):
    mesh = plsc.VectorSubcoreMesh(core_axis_name="c", subcore_axis_name="s",
                                  num_cores=2, num_subcores=16)
    f32 = jnp.float32
    return pl.kernel(
        _sc_sort_body,
        out_type=[
            jax.ShapeDtypeStruct((_L, _C), jnp.int32),
            jax.ShapeDtypeStruct((_L, _C), f32),
            jax.ShapeDtypeStruct((_L, _C), f32),
            jax.ShapeDtypeStruct((_L, _C), f32),
            jax.ShapeDtypeStruct((_L, _C), f32),
            jax.ShapeDtypeStruct((_L, 16), jnp.int32),
        ],
        mesh=mesh,
        scratch_types=[
            pltpu.VMEM((_W,), jnp.int32),
            pltpu.VMEM((_W,), jnp.int32),
            pltpu.VMEM((_W,), jnp.int32),
            pltpu.VMEM((_W,), jnp.int32),
            pltpu.VMEM((_W,), f32),
            pltpu.VMEM((_W,), f32),
            pltpu.VMEM((_W,), f32),
            pltpu.VMEM((_W,), f32),
            pltpu.VMEM((512,), jnp.int32),
            pltpu.VMEM((512,), jnp.int32),
            pltpu.VMEM((_C,), f32),
            pltpu.VMEM((_C,), f32),
            pltpu.VMEM((_C,), f32),
            pltpu.VMEM((_C,), f32),
            pltpu.VMEM((16,), jnp.int32),
        ],
        compiler_params=pltpu.CompilerParams(needs_layout_passes=False),
    )(keys, bx1, by1, bx2, by2)


def _make_nms_body(width, ksel, with_flag):
    def body(scores_ref, x1_ref, y1_ref, x2_ref, y2_ref, *refs):
        if with_flag:
            (cls_out, conf_out, ox1_out, oy1_out, ox2_out, oy2_out,
             flag_out, work_ref, area_ref) = refs
        else:
            (cls_out, conf_out, ox1_out, oy1_out, ox2_out, oy2_out,
             work_ref, area_ref) = refs

        scores = scores_ref[...]
        work_ref[...] = jnp.where(scores > _CONF_T, scores, -jnp.inf)
        x1 = x1_ref[...]
        y1 = y1_ref[...]
        x2 = x2_ref[...]
        y2 = y2_ref[...]
        area_ref[...] = (jnp.maximum(x2 - x1, 0.0)
                         * jnp.maximum(y2 - y1, 0.0))

        iota = lax.broadcasted_iota(jnp.int32, (_L, width), 1)
        lane = lax.broadcasted_iota(jnp.int32, (_L, 1), 0)
        clsvec = (lane % _NCLS + 1).astype(jnp.float32)
        col = lax.broadcasted_iota(jnp.int32, (_L, ksel), 1)

        def step(k, _):
            work = work_ref[...]
            m = jnp.max(work, axis=1, keepdims=True)
            msk = work == m
            idx = jnp.min(jnp.where(msk, iota, width), axis=1, keepdims=True)
            onehot = iota == idx

            x1 = x1_ref[...]
            y1 = y1_ref[...]
            x2 = x2_ref[...]
            y2 = y2_ref[...]
            zero = jnp.zeros_like(x1)
            sx1 = jnp.sum(jnp.where(onehot, x1, zero), axis=1, keepdims=True)
            sy1 = jnp.sum(jnp.where(onehot, y1, zero), axis=1, keepdims=True)
            sx2 = jnp.sum(jnp.where(onehot, x2, zero), axis=1, keepdims=True)
            sy2 = jnp.sum(jnp.where(onehot, y2, zero), axis=1, keepdims=True)

            xi1 = jnp.maximum(sx1, x1)
            yi1 = jnp.maximum(sy1, y1)
            xi2 = jnp.minimum(sx2, x2)
            yi2 = jnp.minimum(sy2, y2)
            inter = (jnp.maximum(xi2 - xi1, 0.0)
                     * jnp.maximum(yi2 - yi1, 0.0))
            a1 = (jnp.maximum(sx2 - sx1, 0.0)
                  * jnp.maximum(sy2 - sy1, 0.0))
            iou = inter / (a1 + area_ref[...] - inter + 1e-8)

            supp = (iou >= _IOU_T) | onehot
            work_ref[...] = jnp.where(supp, -jnp.inf, work)

            ok = m > 0.0
            z1 = jnp.zeros_like(m)
            here = col == k
            for ref, val in ((cls_out, clsvec), (conf_out, m),
                             (ox1_out, sx1), (oy1_out, sy1),
                             (ox2_out, sx2), (oy2_out, sy2)):
                v = jnp.where(ok, val, z1)
                ref[...] = jnp.where(here, v, ref[...])
            if with_flag:
                flag_out[...] = jnp.broadcast_to(
                    jnp.where(ok, z1, z1 + 1.0), (_L, 128))
            return 0

        lax.fori_loop(0, ksel, step, 0)

    return body


def _tc_nms(scores, x1, y1, x2, y2, width, ksel, with_flag):
    f32 = jnp.float32
    out_sd = [jax.ShapeDtypeStruct((_L, ksel), f32)] * 6
    if with_flag:
        out_sd = out_sd + [jax.ShapeDtypeStruct((_L, 128), f32)]
    return pl.pallas_call(
        _make_nms_body(width, ksel, with_flag),
        out_shape=out_sd,
        scratch_shapes=[
            pltpu.VMEM((_L, width), f32),
            pltpu.VMEM((_L, width), f32),
        ],
    )(scores, x1, y1, x2, y2)


def _decode_boxes(y_pred):
    cx = y_pred[..., -12] * y_pred[..., -4] * y_pred[..., -6] + y_pred[..., -8]
    cy = y_pred[..., -11] * y_pred[..., -3] * y_pred[..., -5] + y_pred[..., -7]
    w = jnp.exp(y_pred[..., -10] * y_pred[..., -2]) * y_pred[..., -6]
    h = jnp.exp(y_pred[..., -9] * y_pred[..., -1]) * y_pred[..., -5]
    xmin = (cx - 0.5 * w) * _IMG_W
    ymin = (cy - 0.5 * h) * _IMG_H
    xmax = (cx + 0.5 * w) * _IMG_W
    ymax = (cy + 0.5 * h) * _IMG_H
    return xmin, ymin, xmax, ymax


def kernel(y_pred):
    xmin, ymin, xmax, ymax = _decode_boxes(y_pred)
    confs = y_pred[..., 1:_NCLS + 1]

    pad = _W - _N
    scores = jnp.transpose(confs, (0, 2, 1))
    scores = jnp.pad(scores, ((0, 0), (0, 0), (0, pad))).reshape(_L, _W)
    sbits = lax.bitcast_convert_type(scores, jnp.int32)
    keys = jnp.where(sbits > _KTHR, _KMAX - sbits, _KINACT)

    boxes_b = [jnp.pad(a, ((0, 0), (0, pad)))
               for a in (xmin, ymin, xmax, ymax)]

    skey, sx1, sy1, sx2, sy2, cnt = _sc_sort(keys, *boxes_b)
    sscores = lax.bitcast_convert_type(_KMAX - skey, jnp.float32)

    fast = _tc_nms(sscores, sx1, sy1, sx2, sy2, _C, _KSEL, True)
    fast_rows, flag = fast[:6], fast[6]
    exhausted = flag[:, 0] > 0.0

    flat_conf = fast_rows[1].reshape(_B, _NCLS * _KSEL)
    _, top_idx = lax.top_k(flat_conf, _TOPK)
    v200 = jnp.take_along_axis(flat_conf, top_idx[:, -1:], axis=1)
    v_lane = jnp.repeat(v200[:, 0], _NCLS)

    conf_last = fast_rows[1][:, _KSEL - 1]
    bad_a = (~exhausted) & (conf_last >= v_lane)
    bad_b = exhausted & (cnt[:, 0] > _C)
    need_fb = jnp.any(bad_a | bad_b)

    def fast_path(_):
        stacked = jnp.stack([f.reshape(_B, _NCLS * _KSEL)
                             for f in fast_rows], axis=-1)
        return jnp.take_along_axis(stacked, top_idx[..., None], axis=1)

    def fallback(_):
        def lanes(a):
            return jnp.broadcast_to(
                a[:, None, :], (_B, _NCLS, _W)).reshape(_L, _W)
        full = _tc_nms(scores, *[lanes(a) for a in boxes_b], _W, _KFULL,
                       False)
        fc = full[1].reshape(_B, _NCLS * _KFULL)
        _, ti = lax.top_k(fc, _TOPK)
        stacked = jnp.stack([f.reshape(_B, _NCLS * _KFULL)
                             for f in full], axis=-1)
        return jnp.take_along_axis(stacked, ti[..., None], axis=1)

    return lax.cond(need_fb, fallback, fast_path, None)


# speedup vs baseline: 150.5783x; 1.0049x over previous
import functools

import jax
import jax.numpy as jnp
from jax import lax
from jax.experimental import pallas as pl
from jax.experimental.pallas import tpu as pltpu
from jax.experimental.pallas import tpu_sc as plsc

_CONF_T = 0.01
_IOU_T = 0.45
_TOPK = 200
_KSEL = 48
_KFULL = 200
_B = 4
_N = 8732
_NCLS = 20
_L = _B * _NCLS
_W = 8960
_C = 256
_IMG_H = 300.0
_IMG_W = 300.0

_KMAX = 0x3F7FFFFF
_KTHR = 0x3C23D70A
_KINACT = _KMAX - _KTHR
_NV = _W // 16
_NVC = _C // 16
_NW = 32


def _sc_sort_body(keys_hbm, x1_hbm, y1_hbm, x2_hbm, y2_hbm,
                  okey, ox1, oy1, ox2, oy2, ocnt,
                  kA, iA, kB, iB, vx1, vy1, vx2, vy2,
                  hist, base, s1, s2, s3, s4, scnt):
    cid = lax.axis_index("c")
    sid = lax.axis_index("s")
    wid = sid * 2 + cid

    def process(lane):
        batch = ((lane >= _NCLS).astype(jnp.int32)
                 + (lane >= 2 * _NCLS) + (lane >= 3 * _NCLS))
        pltpu.sync_copy(keys_hbm.at[lane], kA)

        kthr = _KINACT
        lane16 = lax.iota(jnp.int32, 16)

        one16 = jnp.ones((16,), jnp.int32)
        zero16 = jnp.zeros((16,), jnp.int32)

        z16 = jnp.zeros((16,), jnp.int32)
        big16 = jnp.full((16,), jnp.int32(_W + 16))
        bufs = ((kA, iA, kB, iB), (kB, iB, kA, iA))

        def radix_pass(p, shift, nvr, count_active=False, limit=None):
            src_k, src_i, dst_k, dst_i = bufs[p % 2]

            def zbody(j, _):
                hist[pl.ds(j * 16, 16)] = z16
                return 0

            lax.fori_loop(0, 32, zbody, 0)

            def hbody(i, acc):
                kv = src_k[pl.ds(i * 16, 16)]
                dig = (kv >> shift) & 511
                cnt, last = plsc.scan_count(dig)
                plsc.addupdate_scatter(hist, [dig], cnt, mask=last)
                if count_active:
                    acc = acc + jnp.sum(jnp.where(kv < kthr, one16, zero16))
                return acc

            acc = lax.fori_loop(0, nvr, hbody, jnp.int32(0))
            if count_active:
                scnt[...] = jnp.broadcast_to(acc, (16,))
                pltpu.sync_copy(scnt, ocnt.at[lane])

            def sbody(j, carry):
                h = hist[pl.ds(j * 16, 16)]
                c = plsc.cumsum(h)
                base[pl.ds(j * 16, 16)] = c - h + carry
                return carry + jnp.max(c)

            lax.fori_loop(0, 32, sbody, jnp.int32(0))

            limit = None
            m16 = None
            if p == 0:
                def mbody(j, mv):
                    sl = pl.ds(j * 16, 16)
                    e = base[sl] + hist[sl]
                    return jnp.minimum(mv, jnp.where(e >= _C, e, big16))

                mvec = lax.fori_loop(0, 32, mbody, big16)
                m16 = (jnp.min(mvec) + 15) >> 4
                limit = m16 * 16

            def pbody(i, _):
                s = pl.ds(i * 16, 16)
                kv = src_k[s]
                if p == 0:
                    iv = lane16 + i * 16
                else:
                    iv = src_i[s]
                dig = (kv >> shift) & 511
                cnt, last = plsc.scan_count(dig)
                pos = plsc.load_gather(base, [dig]) + cnt - 1
                if limit is None:
                    plsc.store_scatter(dst_k, [pos], kv)
                    plsc.store_scatter(dst_i, [pos], iv)
                else:
                    inreg = pos < limit
                    plsc.store_scatter(dst_k, [pos], kv, mask=inreg)
                    plsc.store_scatter(dst_i, [pos], iv, mask=inreg)
                plsc.addupdate_scatter(base, [dig], cnt, mask=last)
                return 0

            lax.fori_loop(0, nvr, pbody, 0)
            return m16

        m16 = radix_pass(0, 18, _NV, count_active=True)
        radix_pass(1, 0, m16)
        radix_pass(2, 9, m16)
        radix_pass(3, 18, m16)

        pltpu.sync_copy(x1_hbm.at[batch], vx1)
        pltpu.sync_copy(y1_hbm.at[batch], vy1)
        pltpu.sync_copy(x2_hbm.at[batch], vx2)
        pltpu.sync_copy(y2_hbm.at[batch], vy2)

        def gbody(j, _):
            s = pl.ds(j * 16, 16)
            iv = iA[s]
            s1[s] = plsc.load_gather(vx1, [iv])
            s2[s] = plsc.load_gather(vy1, [iv])
            s3[s] = plsc.load_gather(vx2, [iv])
            s4[s] = plsc.load_gather(vy2, [iv])
            return 0

        lax.fori_loop(0, _NVC, gbody, 0)

        pltpu.sync_copy(kA.at[pl.ds(0, _C)], okey.at[lane])
        pltpu.sync_copy(s1, ox1.at[lane])
        pltpu.sync_copy(s2, oy1.at[lane])
        pltpu.sync_copy(s3, ox2.at[lane])
        pltpu.sync_copy(s4, oy2.at[lane])

    for r in range(3):
        lane = wid + _NW * r
        if r < 2:
            process(lane)
        else:
            @pl.when(lane < _L)
            def _():
                process(lane)


@functools.partial(jax.jit, static_argnums=())
def _sc_sort(keys, bx1, by1, bx2, by2):
    mesh = plsc.VectorSubcoreMesh(core_axis_name="c", subcore_axis_name="s",
                                  num_cores=2, num_subcores=16)
    f32 = jnp.float32
    return pl.kernel(
        _sc_sort_body,
        out_type=[
            jax.ShapeDtypeStruct((_L, _C), jnp.int32),
            jax.ShapeDtypeStruct((_L, _C), f32),
            jax.ShapeDtypeStruct((_L, _C), f32),
            jax.ShapeDtypeStruct((_L, _C), f32),
            jax.ShapeDtypeStruct((_L, _C), f32),
            jax.ShapeDtypeStruct((_L, 16), jnp.int32),
        ],
        mesh=mesh,
        scratch_types=[
            pltpu.VMEM((_W,), jnp.int32),
            pltpu.VMEM((_W,), jnp.int32),
            pltpu.VMEM((_W,), jnp.int32),
            pltpu.VMEM((_W,), jnp.int32),
            pltpu.VMEM((_W,), f32),
            pltpu.VMEM((_W,), f32),
            pltpu.VMEM((_W,), f32),
            pltpu.VMEM((_W,), f32),
            pltpu.VMEM((512,), jnp.int32),
            pltpu.VMEM((512,), jnp.int32),
            pltpu.VMEM((_C,), f32),
            pltpu.VMEM((_C,), f32),
            pltpu.VMEM((_C,), f32),
            pltpu.VMEM((_C,), f32),
            pltpu.VMEM((16,), jnp.int32),
        ],
        compiler_params=pltpu.CompilerParams(needs_layout_passes=False),
    )(keys, bx1, by1, bx2, by2)


def _make_nms_body(width, ksel, with_flag):
    def body(scores_ref, x1_ref, y1_ref, x2_ref, y2_ref, *refs):
        if with_flag:
            (cls_out, conf_out, ox1_out, oy1_out, ox2_out, oy2_out,
             flag_out, work_ref, area_ref) = refs
        else:
            (cls_out, conf_out, ox1_out, oy1_out, ox2_out, oy2_out,
             work_ref, area_ref) = refs

        scores = scores_ref[...]
        work_ref[...] = jnp.where(scores > _CONF_T, scores, -jnp.inf)
        x1 = x1_ref[...]
        y1 = y1_ref[...]
        x2 = x2_ref[...]
        y2 = y2_ref[...]
        area_ref[...] = (jnp.maximum(x2 - x1, 0.0)
                         * jnp.maximum(y2 - y1, 0.0))

        iota = lax.broadcasted_iota(jnp.int32, (_L, width), 1)
        lane = lax.broadcasted_iota(jnp.int32, (_L, 1), 0)
        clsvec = (lane % _NCLS + 1).astype(jnp.float32)
        col = lax.broadcasted_iota(jnp.int32, (_L, ksel), 1)

        def step(k, _):
            work = work_ref[...]
            m = jnp.max(work, axis=1, keepdims=True)
            msk = work == m
            idx = jnp.min(jnp.where(msk, iota, width), axis=1, keepdims=True)
            onehot = iota == idx

            x1 = x1_ref[...]
            y1 = y1_ref[...]
            x2 = x2_ref[...]
            y2 = y2_ref[...]
            zero = jnp.zeros_like(x1)
            sx1 = jnp.sum(jnp.where(onehot, x1, zero), axis=1, keepdims=True)
            sy1 = jnp.sum(jnp.where(onehot, y1, zero), axis=1, keepdims=True)
            sx2 = jnp.sum(jnp.where(onehot, x2, zero), axis=1, keepdims=True)
            sy2 = jnp.sum(jnp.where(onehot, y2, zero), axis=1, keepdims=True)

            xi1 = jnp.maximum(sx1, x1)
            yi1 = jnp.maximum(sy1, y1)
            xi2 = jnp.minimum(sx2, x2)
            yi2 = jnp.minimum(sy2, y2)
            inter = (jnp.maximum(xi2 - xi1, 0.0)
                     * jnp.maximum(yi2 - yi1, 0.0))
            a1 = (jnp.maximum(sx2 - sx1, 0.0)
                  * jnp.maximum(sy2 - sy1, 0.0))
            iou = inter / (a1 + area_ref[...] - inter + 1e-8)

            supp = (iou >= _IOU_T) | onehot
            work_ref[...] = jnp.where(supp, -jnp.inf, work)

            ok = m > 0.0
            z1 = jnp.zeros_like(m)
            here = col == k
            for ref, val in ((cls_out, clsvec), (conf_out, m),
                             (ox1_out, sx1), (oy1_out, sy1),
                             (ox2_out, sx2), (oy2_out, sy2)):
                v = jnp.where(ok, val, z1)
                ref[...] = jnp.where(here, v, ref[...])
            if with_flag:
                flag_out[...] = jnp.broadcast_to(
                    jnp.where(ok, z1, z1 + 1.0), (_L, 128))
            return 0

        lax.fori_loop(0, ksel, step, 0, unroll=4)

    return body


def _tc_nms(scores, x1, y1, x2, y2, width, ksel, with_flag):
    f32 = jnp.float32
    out_sd = [jax.ShapeDtypeStruct((_L, ksel), f32)] * 6
    if with_flag:
        out_sd = out_sd + [jax.ShapeDtypeStruct((_L, 128), f32)]
    return pl.pallas_call(
        _make_nms_body(width, ksel, with_flag),
        out_shape=out_sd,
        scratch_shapes=[
            pltpu.VMEM((_L, width), f32),
            pltpu.VMEM((_L, width), f32),
        ],
    )(scores, x1, y1, x2, y2)


def _decode_boxes(y_pred):
    cx = y_pred[..., -12] * y_pred[..., -4] * y_pred[..., -6] + y_pred[..., -8]
    cy = y_pred[..., -11] * y_pred[..., -3] * y_pred[..., -5] + y_pred[..., -7]
    w = jnp.exp(y_pred[..., -10] * y_pred[..., -2]) * y_pred[..., -6]
    h = jnp.exp(y_pred[..., -9] * y_pred[..., -1]) * y_pred[..., -5]
    xmin = (cx - 0.5 * w) * _IMG_W
    ymin = (cy - 0.5 * h) * _IMG_H
    xmax = (cx + 0.5 * w) * _IMG_W
    ymax = (cy + 0.5 * h) * _IMG_H
    return xmin, ymin, xmax, ymax


def kernel(y_pred):
    xmin, ymin, xmax, ymax = _decode_boxes(y_pred)
    confs = y_pred[..., 1:_NCLS + 1]

    pad = _W - _N
    scores = jnp.transpose(confs, (0, 2, 1))
    scores = jnp.pad(scores, ((0, 0), (0, 0), (0, pad))).reshape(_L, _W)
    sbits = lax.bitcast_convert_type(scores, jnp.int32)
    keys = jnp.where(sbits > _KTHR, _KMAX - sbits, _KINACT)

    boxes_b = [jnp.pad(a, ((0, 0), (0, pad)))
               for a in (xmin, ymin, xmax, ymax)]

    skey, sx1, sy1, sx2, sy2, cnt = _sc_sort(keys, *boxes_b)
    sscores = lax.bitcast_convert_type(_KMAX - skey, jnp.float32)

    fast = _tc_nms(sscores, sx1, sy1, sx2, sy2, _C, _KSEL, True)
    fast_rows, flag = fast[:6], fast[6]
    exhausted = flag[:, 0] > 0.0

    flat_conf = fast_rows[1].reshape(_B, _NCLS * _KSEL)
    _, top_idx = lax.top_k(flat_conf, _TOPK)
    v200 = jnp.take_along_axis(flat_conf, top_idx[:, -1:], axis=1)
    v_lane = jnp.repeat(v200[:, 0], _NCLS)

    conf_last = fast_rows[1][:, _KSEL - 1]
    bad_a = (~exhausted) & (conf_last >= v_lane)
    bad_b = exhausted & (cnt[:, 0] > _C)
    need_fb = jnp.any(bad_a | bad_b)

    def fast_path(_):
        stacked = jnp.stack([f.reshape(_B, _NCLS * _KSEL)
                             for f in fast_rows], axis=-1)
        return jnp.take_along_axis(stacked, top_idx[..., None], axis=1)

    def fallback(_):
        def lanes(a):
            return jnp.broadcast_to(
                a[:, None, :], (_B, _NCLS, _W)).reshape(_L, _W)
        full = _tc_nms(scores, *[lanes(a) for a in boxes_b], _W, _KFULL,
                       False)
        fc = full[1].reshape(_B, _NCLS * _KFULL)
        _, ti = lax.top_k(fc, _TOPK)
        stacked = jnp.stack([f.reshape(_B, _NCLS * _KFULL)
                             for f in full], axis=-1)
        return jnp.take_along_axis(stacked, ti[..., None], axis=1)

    return lax.cond(need_fb, fallback, fast_path, None)
